# baseline (speedup 1.0000x reference)
import functools

import jax
import jax.numpy as jnp
from jax import lax
from jax.experimental import pallas as pl
from jax.experimental.pallas import tpu as pltpu
from jax.experimental.pallas import tpu_sc as plsc

NUM_EMBEDDINGS = 1000000
EMBEDDING_DIM = 64
BATCH = 16384

NUM_CORES = 2
NUM_SUBCORES = 16
NW = NUM_CORES * NUM_SUBCORES
LANES = 16
W = 256
NFULL = 999936 // W
LAST_W = NUM_EMBEDDINGS - NFULL * W
T_MAX = NFULL // NW + 1
CAP = 32
CAP_ALL = 640
NCHUNK = CAP_ALL // 128
OUT_ROWS = BATCH + 128
OUT_W = 2 * EMBEDDING_DIM
IDX_CHUNK = 2048


def _build():
    mesh = plsc.VectorSubcoreMesh(core_axis_name="c", subcore_axis_name="s")

    @functools.partial(
        pl.kernel,
        mesh=mesh,
        out_type=jax.ShapeDtypeStruct((OUT_ROWS, OUT_W), jnp.float32),
        scratch_types=[
            pltpu.VMEM((IDX_CHUNK,), jnp.int32),
            pltpu.VMEM((T_MAX * CAP,), jnp.int32),
            pltpu.VMEM((((T_MAX + LANES - 1) // LANES) * LANES,),
                       jnp.int32),
            pltpu.VMEM((2, EMBEDDING_DIM, W), jnp.float32),
            pltpu.VMEM((EMBEDDING_DIM, LAST_W), jnp.float32),
            pltpu.VMEM((CAP_ALL, OUT_W), jnp.float32),
            pltpu.VMEM((NCHUNK, 128), jnp.int32),
            pltpu.SemaphoreType.DMA,
        ],
        compiler_params=pltpu.CompilerParams(use_tc_tiling_on_sc=True,
                                             needs_layout_passes=False),
    )
    def scan_kernel(idx_hbm, tableT_hbm, tail_hbm, out_hbm,
                    idxb, bpack, bcnt, slab, tailv, hv, spv, sem):
        wid = lax.axis_index("s") * NUM_CORES + lax.axis_index("c")
        lane = lax.iota(jnp.int32, LANES)
        lane0 = lane == 0

        def splat(x):
            return jnp.broadcast_to(x, (LANES,))

        def extract(vec, sel_mask):
            return splat(jnp.max(jnp.where(sel_mask, vec, -1)))

        for i in range((T_MAX + LANES - 1) // LANES):
            bcnt[pl.ds(i * LANES, LANES)] = jnp.zeros((LANES,), jnp.int32)
        for c in range(NCHUNK):
            for i in range(128 // LANES):
                spv[c, pl.ds(i * LANES, LANES)] = BATCH + i * LANES + lane

        def pf_body(q, carry):
            v = plsc.load_gather(
                idxb, [(q & (IDX_CHUNK // LANES - 1)) * LANES + lane])
            gstep = v >> 8
            off = v & (W - 1)
            mine = (gstep & (NW - 1)) == wid
            tloc = gstep >> 5
            pos = q * LANES + lane
            packed = (pos << 8) | off
            n = jnp.sum(jnp.where(mine, 1, 0))

            def hit_body(j, m):
                f = plsc.all_reduce_ffs(m)
                isf = lane == f
                ts = extract(tloc, isf)
                pk = extract(packed, isf)
                cnt = plsc.load_gather(bcnt, [ts])
                slot = ts * CAP + cnt
                plsc.store_scatter(bpack, [slot], pk, mask=lane0)
                plsc.store_scatter(bcnt, [ts], cnt + 1, mask=lane0)
                return jnp.logical_and(m, jnp.logical_not(isf))

            lax.fori_loop(0, n, hit_body, mine)
            return carry

        for r in range(BATCH // IDX_CHUNK):
            pltpu.sync_copy(idx_hbm.at[pl.ds(r * IDX_CHUNK, IDX_CHUNK)], idxb)
            lax.fori_loop(r * (IDX_CHUNK // LANES),
                          (r + 1) * (IDX_CHUNK // LANES), pf_body, 0)

        T = jnp.where(wid < NFULL - (T_MAX - 1) * NW, T_MAX, T_MAX - 1)

        def process_window(t, hcnt, src, pfx):
            cntv = plsc.load_gather(bcnt, [splat(t)])
            n_t = jnp.max(cntv)

            def hb(j, hc):
                pk = plsc.load_gather(bpack, [splat(t * CAP + j)])
                off = pk & (W - 1)
                pos = pk >> 8
                slot = hc + splat(j)
                for cq in range(EMBEDDING_DIM // LANES):
                    dv = cq * LANES + lane
                    val = plsc.load_gather(src, pfx + [dv, off])
                    plsc.store_scatter(hv, [slot, dv], val)
                plsc.store_scatter(spv, [slot >> 7, slot & 127], pos,
                                   mask=lane0)
                return hc

            lax.fori_loop(0, n_t, hb, hcnt)
            return hcnt + splat(n_t)

        def step_body(t, hcnt):
            par = t & 1

            @pl.when(jnp.logical_and(t + 1 < T, par == 0))
            def _():
                c2 = (wid + (t + 1) * NW) * W
                pltpu.async_copy(tableT_hbm.at[:, pl.ds(c2, W)],
                                 slab.at[1], sem)

            @pl.when(jnp.logical_and(t + 1 < T, par == 1))
            def _():
                c2 = (wid + (t + 1) * NW) * W
                pltpu.async_copy(tableT_hbm.at[:, pl.ds(c2, W)],
                                 slab.at[0], sem)

            pltpu.make_async_copy(tableT_hbm.at[:, pl.ds(0, W)],
                                  slab.at[0], sem).wait()
            return process_window(t, hcnt, slab, [splat(par)])

        pltpu.async_copy(tableT_hbm.at[:, pl.ds(wid * W, W)], slab.at[0], sem)
        hcnt = lax.fori_loop(0, T, step_body, splat(0))

        @pl.when(wid == NFULL % NW)
        def _():
            pltpu.sync_copy(tail_hbm, tailv)
            process_window(NFULL // NW, hcnt, tailv, [])

        copies = [
            pltpu.async_copy(hv.at[pl.ds(c * 128, 128)],
                             out_hbm.at[spv.at[c]], sem)
            for c in range(NCHUNK)
        ]
        for cp in copies:
            cp.wait()

    return scan_kernel


_sc_scan = _build()


def kernel(input_, weight):
    table_t = weight.T
    tail = table_t[:, NFULL * W:]
    out = _sc_scan(input_.astype(jnp.int32), table_t, tail)
    return out[:BATCH, :EMBEDDING_DIM]
